# Initial kernel scaffold; baseline (speedup 1.0000x reference)
import functools

import jax
import jax.numpy as jnp
from jax import lax
from jax.experimental import pallas as pl
from jax.experimental.pallas import tpu as pltpu
from jax.experimental.pallas import tpu_sc as plsc


def _make_emb(n_groups, group, D, nw, groups_per_chunk):
    gpw = n_groups // nw
    n_chunks = gpw // groups_per_chunk
    G = groups_per_chunk

    mesh = plsc.VectorSubcoreMesh(core_axis_name="c", subcore_axis_name="s")

    @functools.partial(
        pl.kernel,
        mesh=mesh,
        out_type=jax.ShapeDtypeStruct((n_groups, group, D), jnp.float32),
        scratch_types=[
            pltpu.VMEM((G, group), jnp.int32),
            pltpu.VMEM((G, group, D), jnp.float32),
            pltpu.SemaphoreType.DMA,
        ],
    )
    def emb(idx_hbm, table_hbm, out_hbm, idx_v, rows_v, sem):
        nc = 2
        wid = lax.axis_index("s") * nc + lax.axis_index("c")
        base = wid * gpw

        def body(i, _):
            g0 = base + i * G
            pltpu.sync_copy(idx_hbm.at[pl.ds(g0, G)], idx_v)
            copies = [
                pltpu.async_copy(table_hbm.at[idx_v.at[j]], rows_v.at[j], sem)
                for j in range(G)
            ]
            for c in copies:
                c.wait()
            pltpu.sync_copy(rows_v, out_hbm.at[pl.ds(g0, G)])
            return ()

        lax.fori_loop(0, n_chunks, body, ())

    return emb


def kernel(tokens, table):
    B, T = tokens.shape
    V, D = table.shape
    N = B * T
    group = 128
    n_groups = N // group
    idx2d = tokens.reshape(n_groups, group).astype(jnp.int32)
    emb = _make_emb(n_groups, group, D, nw=32, groups_per_chunk=8)
    out = emb(idx2d, table)
    return out.reshape(B, T, D)


# speedup vs baseline: 4.1377x; 4.1377x over previous
import functools

import jax
import jax.numpy as jnp
from jax import lax
from jax.experimental import pallas as pl
from jax.experimental.pallas import tpu as pltpu
from jax.experimental.pallas import tpu_sc as plsc


def _make_emb(n_groups, group, D, nw, groups_per_chunk):
    gpw = n_groups // nw
    n_chunks = gpw // groups_per_chunk
    G = groups_per_chunk

    mesh = plsc.VectorSubcoreMesh(core_axis_name="c", subcore_axis_name="s")

    @functools.partial(
        pl.kernel,
        mesh=mesh,
        out_type=jax.ShapeDtypeStruct((n_groups, group, D), jnp.float32),
        scratch_types=[
            pltpu.VMEM((G, group), jnp.int32),
            pltpu.VMEM((G, group, D), jnp.float32),
            pltpu.SemaphoreType.DMA,
        ],
        compiler_params=pltpu.CompilerParams(use_tc_tiling_on_sc=False),
    )
    def emb(idx_hbm, table_hbm, out_hbm, idx_v, rows_v, sem):
        nc = 2
        wid = lax.axis_index("s") * nc + lax.axis_index("c")
        base = wid * gpw

        def body(i, _):
            g0 = base + i * G
            pltpu.sync_copy(idx_hbm.at[pl.ds(g0, G)], idx_v)
            copies = [
                pltpu.async_copy(table_hbm.at[idx_v.at[j]], rows_v.at[j], sem)
                for j in range(G)
            ]
            for c in copies:
                c.wait()
            pltpu.sync_copy(rows_v, out_hbm.at[pl.ds(g0, G)])
            return ()

        lax.fori_loop(0, n_chunks, body, ())

    return emb


def kernel(tokens, table):
    B, T = tokens.shape
    V, D = table.shape
    N = B * T
    group = 128
    n_groups = N // group
    idx2d = tokens.reshape(n_groups, group).astype(jnp.int32)
    emb = _make_emb(n_groups, group, D, nw=32, groups_per_chunk=8)
    out = emb(idx2d, table)
    return out.reshape(B, T, D)


# speedup vs baseline: 4.1959x; 1.0141x over previous
import functools

import jax
import jax.numpy as jnp
from jax import lax
from jax.experimental import pallas as pl
from jax.experimental.pallas import tpu as pltpu
from jax.experimental.pallas import tpu_sc as plsc


def _make_emb(n_groups, group, D, nw, G):
    gpw = n_groups // nw
    n_chunks = gpw // G
    n2 = n_chunks // 2

    mesh = plsc.VectorSubcoreMesh(core_axis_name="c", subcore_axis_name="s")

    @functools.partial(
        pl.kernel,
        mesh=mesh,
        out_type=jax.ShapeDtypeStruct((n_groups, group, D), jnp.float32),
        scratch_types=[
            pltpu.VMEM((G, group), jnp.int32),
            pltpu.VMEM((G, group), jnp.int32),
            pltpu.VMEM((G, group, D), jnp.float32),
            pltpu.VMEM((G, group, D), jnp.float32),
            pltpu.SemaphoreType.DMA,
            pltpu.SemaphoreType.DMA,
            pltpu.SemaphoreType.DMA,
            pltpu.SemaphoreType.DMA,
        ],
        compiler_params=pltpu.CompilerParams(use_tc_tiling_on_sc=False),
    )
    def emb(idx_hbm, table_hbm, out_hbm, idx_a, idx_b, rows_a, rows_b,
            gs_a, gs_b, ws_a, ws_b):
        nc = 2
        wid = lax.axis_index("s") * nc + lax.axis_index("c")
        base = wid * gpw

        def stage(idx_v, c):
            pltpu.sync_copy(idx_hbm.at[pl.ds(base + c * G, G)], idx_v)

        def fire_g(idx_v, rows_v, sem):
            for j in range(G):
                pltpu.async_copy(table_hbm.at[idx_v.at[j]], rows_v.at[j], sem)

        def wait_g(idx_v, rows_v, sem):
            for j in range(G):
                pltpu.make_async_copy(
                    table_hbm.at[idx_v.at[j]], rows_v.at[j], sem).wait()

        def fire_w(rows_v, c, sem):
            pltpu.async_copy(rows_v, out_hbm.at[pl.ds(base + c * G, G)], sem)

        def wait_w(rows_v, c, sem):
            pltpu.make_async_copy(
                rows_v, out_hbm.at[pl.ds(base + c * G, G)], sem).wait()

        stage(idx_a, 0)
        fire_g(idx_a, rows_a, gs_a)

        def body(r2, _):
            c = 2 * r2
            wait_g(idx_a, rows_a, gs_a)
            fire_w(rows_a, c, ws_a)

            @pl.when(r2 > 0)
            def _():
                wait_w(rows_b, c - 1, ws_b)

            stage(idx_b, c + 1)
            fire_g(idx_b, rows_b, gs_b)
            wait_g(idx_b, rows_b, gs_b)
            fire_w(rows_b, c + 1, ws_b)
            wait_w(rows_a, c, ws_a)

            @pl.when(r2 < n2 - 1)
            def _():
                stage(idx_a, c + 2)
                fire_g(idx_a, rows_a, gs_a)
            return ()

        lax.fori_loop(0, n2, body, ())
        wait_w(rows_b, n_chunks - 1, ws_b)

    return emb


def kernel(tokens, table):
    B, T = tokens.shape
    V, D = table.shape
    N = B * T
    group = 128
    n_groups = N // group
    idx2d = tokens.reshape(n_groups, group).astype(jnp.int32)
    emb = _make_emb(n_groups, group, D, nw=32, G=5)
    out = emb(idx2d, table)
    return out.reshape(B, T, D)


# speedup vs baseline: 4.2167x; 1.0050x over previous
import functools

import jax
import jax.numpy as jnp
from jax import lax
from jax.experimental import pallas as pl
from jax.experimental.pallas import tpu as pltpu
from jax.experimental.pallas import tpu_sc as plsc

_SPLITS = ((0, 104), (104, 96))


def _make_emb(B, T, D, nw, G):
    rpw = B // nw
    n_chunks = rpw // G
    n2 = n_chunks // 2

    mesh = plsc.VectorSubcoreMesh(core_axis_name="c", subcore_axis_name="s")

    @functools.partial(
        pl.kernel,
        mesh=mesh,
        out_type=jax.ShapeDtypeStruct((B, T, D), jnp.float32),
        scratch_types=[
            pltpu.VMEM((G, T), jnp.int32),
            pltpu.VMEM((G, T), jnp.int32),
            pltpu.VMEM((G, T, D), jnp.float32),
            pltpu.VMEM((G, T, D), jnp.float32),
            pltpu.SemaphoreType.DMA,
            pltpu.SemaphoreType.DMA,
            pltpu.SemaphoreType.DMA,
            pltpu.SemaphoreType.DMA,
        ],
        compiler_params=pltpu.CompilerParams(use_tc_tiling_on_sc=False),
    )
    def emb(tok_hbm, table_hbm, out_hbm, idx_a, idx_b, rows_a, rows_b,
            gs_a, gs_b, ws_a, ws_b):
        nc = 2
        wid = lax.axis_index("s") * nc + lax.axis_index("c")
        base = wid * rpw

        def stage(idx_v, c):
            pltpu.sync_copy(tok_hbm.at[pl.ds(base + c * G, G)], idx_v)

        def fire_g(idx_v, rows_v, sem):
            for j in range(G):
                for o, l in _SPLITS:
                    pltpu.async_copy(
                        table_hbm.at[idx_v.at[j, pl.ds(o, l)]],
                        rows_v.at[j, pl.ds(o, l)], sem)

        def wait_g(idx_v, rows_v, sem):
            for j in range(G):
                for o, l in _SPLITS:
                    pltpu.make_async_copy(
                        table_hbm.at[idx_v.at[j, pl.ds(o, l)]],
                        rows_v.at[j, pl.ds(o, l)], sem).wait()

        def fire_w(rows_v, c, sem):
            pltpu.async_copy(rows_v, out_hbm.at[pl.ds(base + c * G, G)], sem)

        def wait_w(rows_v, c, sem):
            pltpu.make_async_copy(
                rows_v, out_hbm.at[pl.ds(base + c * G, G)], sem).wait()

        stage(idx_a, 0)
        fire_g(idx_a, rows_a, gs_a)

        def body(r2, _):
            c = 2 * r2
            wait_g(idx_a, rows_a, gs_a)
            fire_w(rows_a, c, ws_a)

            @pl.when(r2 > 0)
            def _():
                wait_w(rows_b, c - 1, ws_b)

            stage(idx_b, c + 1)
            fire_g(idx_b, rows_b, gs_b)
            wait_g(idx_b, rows_b, gs_b)
            fire_w(rows_b, c + 1, ws_b)
            wait_w(rows_a, c, ws_a)

            @pl.when(r2 < n2 - 1)
            def _():
                stage(idx_a, c + 2)
                fire_g(idx_a, rows_a, gs_a)
            return ()

        lax.fori_loop(0, n2, body, ())
        wait_w(rows_b, n_chunks - 1, ws_b)

    return emb


def kernel(tokens, table):
    B, T = tokens.shape
    V, D = table.shape
    emb = _make_emb(B, T, D, nw=32, G=4)
    return emb(tokens.astype(jnp.int32), table)


# speedup vs baseline: 6.0877x; 1.4437x over previous
import functools

import jax
import jax.numpy as jnp
from jax import lax
from jax.experimental import pallas as pl
from jax.experimental.pallas import tpu as pltpu
from jax.experimental.pallas import tpu_sc as plsc


def _make_emb(B, T, V, D, nw):
    cols_per_w = D // nw
    n2 = T // 2

    mesh = plsc.VectorSubcoreMesh(core_axis_name="c", subcore_axis_name="s")

    @functools.partial(
        pl.kernel,
        mesh=mesh,
        out_type=jax.ShapeDtypeStruct((T, D, B), jnp.float32),
        scratch_types=[
            pltpu.VMEM((V,), jnp.float32),
            pltpu.VMEM((B,), jnp.int32),
            pltpu.VMEM((B,), jnp.int32),
            pltpu.VMEM((B,), jnp.float32),
            pltpu.VMEM((B,), jnp.float32),
            pltpu.SemaphoreType.DMA,
            pltpu.SemaphoreType.DMA,
            pltpu.SemaphoreType.DMA,
            pltpu.SemaphoreType.DMA,
        ],
        compiler_params=pltpu.CompilerParams(needs_layout_passes=False),
    )
    def emb(tok_hbm, tab_hbm, out_hbm, col, ia, ib, sa, sb,
            sia, sib, swa, swb):
        nc = 2
        wid = lax.axis_index("s") * nc + lax.axis_index("c")

        def gather(idx_v, stage_v):
            def gb(g, _):
                for u in range(8):
                    off = g * 128 + u * 16
                    iv = idx_v[pl.ds(off, 16)]
                    stage_v[pl.ds(off, 16)] = plsc.load_gather(col, [iv])
                return ()
            lax.fori_loop(0, B // 128, gb, ())

        for p in range(cols_per_w):
            d = wid * cols_per_w + p
            pltpu.sync_copy(tab_hbm.at[d], col)

            pltpu.sync_copy(tok_hbm.at[0], ia)
            pltpu.async_copy(tok_hbm.at[1], ib, sib)

            def body(q, _):
                t = 2 * q

                @pl.when(q > 0)
                def _():
                    pltpu.make_async_copy(tok_hbm.at[t], ia, sia).wait()
                    pltpu.make_async_copy(sa, out_hbm.at[t - 2, d], swa).wait()

                gather(ia, sa)
                pltpu.async_copy(sa, out_hbm.at[t, d], swa)

                @pl.when(q < n2 - 1)
                def _():
                    pltpu.async_copy(tok_hbm.at[t + 2], ia, sia)

                pltpu.make_async_copy(tok_hbm.at[t + 1], ib, sib).wait()

                @pl.when(q > 0)
                def _():
                    pltpu.make_async_copy(sb, out_hbm.at[t - 1, d], swb).wait()

                gather(ib, sb)
                pltpu.async_copy(sb, out_hbm.at[t + 1, d], swb)

                @pl.when(q < n2 - 1)
                def _():
                    pltpu.async_copy(tok_hbm.at[t + 3], ib, sib)
                return ()

            lax.fori_loop(0, n2, body, ())
            pltpu.make_async_copy(sa, out_hbm.at[T - 2, d], swa).wait()
            pltpu.make_async_copy(sb, out_hbm.at[T - 1, d], swb).wait()

    return emb


def kernel(tokens, table):
    B, T = tokens.shape
    V, D = table.shape
    emb = _make_emb(B, T, V, D, nw=32)
    out = emb(tokens.T.astype(jnp.int32), table.T)
    return out.transpose(2, 0, 1)
